# baseline (speedup 1.0000x reference)
import functools

import jax
import jax.numpy as jnp
from jax import lax
from jax.experimental import pallas as pl
from jax.experimental.pallas import tpu as pltpu
from jax.experimental.pallas import tpu_sc as plsc

VOCAB = 100000
HIDDEN = 128
BATCH = 4096
SEQ = 50
EPS = 1e-3
LANES = 16
NH = HIDDEN // LANES

_info = plsc.get_sparse_core_info()
NC, NS = _info.num_cores, _info.num_subcores
NW = NC * NS
BPW = BATCH // NW

CHUNK = 128
NCHUNK = BPW * SEQ // CHUNK
PERIOD = 25
NBUF = 5
ROWS_PER_PERIOD = PERIOD * CHUNK // SEQ

_mesh = plsc.VectorSubcoreMesh(core_axis_name="c", subcore_axis_name="s")


@functools.partial(
    pl.kernel,
    mesh=_mesh,
    out_type=jax.ShapeDtypeStruct((BATCH, HIDDEN), jnp.float32),
    scratch_types=[
        pltpu.VMEM((NCHUNK, CHUNK), jnp.int32),
    ] + [pltpu.VMEM((CHUNK, HIDDEN), jnp.float32)] * NBUF
      + [pltpu.VMEM((BPW, HIDDEN), jnp.float32)]
      + [pltpu.SemaphoreType.DMA] * NBUF,
)
def _pool(idx_hbm, table_hbm, out_hbm, idx_v, *rest):
    bufs = rest[:NBUF]
    out_v = rest[NBUF]
    sems = rest[NBUF + 1:]
    wid = lax.axis_index("s") * NC + lax.axis_index("c")
    base = wid * BPW
    pltpu.sync_copy(idx_hbm.at[wid], idx_v)

    inv = jnp.float32(1.0 / SEQ)
    zeros = tuple(jnp.zeros((LANES,), jnp.float32) for _ in range(NH))

    def addrange(buf, p, n, acc):
        def sbody(s, a):
            return tuple(a[h] + buf[p + s, pl.ds(LANES * h, LANES)]
                         for h in range(NH))
        return lax.fori_loop(0, n, sbody, acc)

    def flush(row, acc):
        for h in range(NH):
            out_v[row, pl.ds(LANES * h, LANES)] = acc[h] * inv

    for k in range(NBUF):
        pltpu.async_copy(table_hbm.at[idx_v.at[k]], bufs[k], sems[k])

    def body(m, _):
        rbase = m * ROWS_PER_PERIOD
        acc = zeros
        for k in range(PERIOD):
            c = m * PERIOD + k
            buf = bufs[k % NBUF]
            sem = sems[k % NBUF]
            pltpu.make_async_copy(table_hbm.at[idx_v.at[c]], buf, sem).wait()

            start = CHUNK * k
            p = 0
            r = start // SEQ
            off = start % SEQ
            if off:
                n = SEQ - off
                acc = addrange(buf, p, n, acc)
                flush(rbase + r, acc)
                acc = zeros
                p += n
                r += 1
            while CHUNK - p >= SEQ:
                acc = addrange(buf, p, SEQ, acc)
                flush(rbase + r, acc)
                acc = zeros
                p += SEQ
                r += 1
            if p < CHUNK:
                acc = addrange(buf, p, CHUNK - p, acc)

            @pl.when(c + NBUF < NCHUNK)
            def _():
                pltpu.async_copy(
                    table_hbm.at[idx_v.at[c + NBUF]], buf, sem)
        return 0

    lax.fori_loop(0, NCHUNK // PERIOD, body, 0)

    pltpu.sync_copy(out_v, out_hbm.at[pl.ds(base, BPW)])


def _ln_body(y_ref, bias_ref, gamma_ref, beta_ref, o_ref):
    x = y_ref[...] + bias_ref[...]
    mu = jnp.mean(x, axis=0, keepdims=True)
    d = x - mu
    var = jnp.mean(d * d, axis=0, keepdims=True)
    o_ref[...] = d * lax.rsqrt(var + EPS) * gamma_ref[...] + beta_ref[...]


_ln = pl.pallas_call(
    _ln_body,
    out_shape=jax.ShapeDtypeStruct((BATCH, HIDDEN), jnp.float32),
)


def kernel(inputs, table, bias, gamma, beta):
    y = _pool(inputs.reshape(NW, NCHUNK, CHUNK), table)
    return _ln(
        y,
        bias.reshape(1, HIDDEN),
        gamma.reshape(BATCH, 1),
        beta.reshape(BATCH, 1),
    )
